# Initial kernel scaffold; baseline (speedup 1.0000x reference)
import jax
import jax.numpy as jnp
from jax.experimental import pallas as pl


def kernel(X, W_base, b_base, W_deep, b_deep, reg_W, train_edges, y_train, sorted_train_edges, surrogates, y):
    raise NotImplementedError("write your pallas kernel here")


# speedup vs baseline: 3.1068x; 3.1068x over previous
import functools

import jax
import jax.numpy as jnp
import numpy as np
from jax import lax
from jax.experimental import pallas as pl
from jax.experimental.pallas import tpu as pltpu
from jax.experimental.pallas import tpu_sc as plsc

N = 10000
DF = 128
NLAB = 3
H = 64
E = 320000
EP = 100000
NL = NLAB * EP

NC, NS = 2, 16
NW = NC * NS

NP = 10112
RPT = NP // NS
DUMMY = N

CHUNK = 128
CPT = 80
EPT = CPT * CHUNK
E_PAD = NW * EPT
EROWS = E_PAD // CHUNK

Q = 9600
NLP = NW * Q
LB = 640
NBLK = Q // LB
NCH = LB // 16

RB = 400
GRID = N // RB

_f32 = jnp.float32


def _mesh():
    return plsc.VectorSubcoreMesh(core_axis_name="c", subcore_axis_name="s",
                                  num_cores=NC, num_subcores=NS)


def _tc0(x_ref, ws_ref, wn_ref, xws_ref, xwn_ref):
    x = x_ref[...]
    dot = functools.partial(jnp.dot, preferred_element_type=_f32,
                            precision=lax.Precision.HIGHEST)
    xws_ref[...] = dot(x, ws_ref[...])
    xwn_ref[...] = dot(x, wn_ref[...])


def _sc_edges(with_lab):
    scratch = [
        pltpu.VMEM((CPT, CHUNK), jnp.int32),
        pltpu.VMEM((CPT, CHUNK), jnp.int32),
        pltpu.VMEM((CHUNK, H), _f32),
        pltpu.VMEM_SHARED((NP, H), _f32),
    ]
    out = [jax.ShapeDtypeStruct((NC, NP, H), _f32)]
    if with_lab:
        scratch += [
            pltpu.VMEM((CPT, CHUNK), jnp.int32),
            pltpu.VMEM((CHUNK, 16), _f32),
            pltpu.VMEM_SHARED((NP, 16), _f32),
        ]
        out += [jax.ShapeDtypeStruct((NC, NP, 16), _f32)]

    @functools.partial(pl.kernel, out_type=out, mesh=_mesh(),
                       scratch_types=scratch,
                       compiler_params=pltpu.CompilerParams(
                           use_tc_tiling_on_sc=False, needs_layout_passes=False))
    def k(*refs):
        if with_lab:
            (tab_h, ltab_h, src_h, dst_h, y_h, z64_h, z16_h,
             s_h, dl_h, src_v, dst_v, rows_v, acc,
             y_v, lab_v, acc16) = refs
        else:
            (tab_h, src_h, dst_h, z64_h,
             s_h, src_v, dst_v, rows_v, acc) = refs
        cid = lax.axis_index("c")
        sid = lax.axis_index("s")
        wid = sid * NC + cid
        rows = pl.ds(sid * RPT, RPT)
        pltpu.sync_copy(z64_h, acc.at[rows])
        if with_lab:
            pltpu.sync_copy(z16_h, acc16.at[rows])
        erows = pl.ds(wid * CPT, CPT)
        pltpu.sync_copy(src_h.at[erows], src_v)
        pltpu.sync_copy(dst_h.at[erows], dst_v)
        if with_lab:
            pltpu.sync_copy(y_h.at[erows], y_v)
        plsc.subcore_barrier()

        def body(j, carry):
            pltpu.sync_copy(tab_h.at[src_v.at[j]], rows_v)
            pltpu.sync_copy(rows_v, acc.at[dst_v.at[j]], add=True)
            if with_lab:
                pltpu.sync_copy(ltab_h.at[y_v.at[j]], lab_v)
                pltpu.sync_copy(lab_v, acc16.at[dst_v.at[j]], add=True)
            return carry

        lax.fori_loop(0, CPT, body, 0)
        plsc.subcore_barrier()
        pltpu.sync_copy(acc.at[rows], s_h.at[cid, rows])
        if with_lab:
            pltpu.sync_copy(acc16.at[rows], dl_h.at[cid, rows])

    return k


def _tc1(xws_ref, s1_ref, dl_ref, wbl_ref, bb_ref, wds_ref, wdn_ref,
         wdl_ref, bd_ref, h1n_ref, pre2_ref, dinv_ref):
    dot = functools.partial(jnp.dot, preferred_element_type=_f32,
                            precision=lax.Precision.HIGHEST)
    s1p = s1_ref[...]
    dlp = dl_ref[...]
    s1 = s1p[0] + s1p[1]
    dl = dlp[0] + dlp[1]
    di = 1.0 / jnp.maximum(dl[:, 0:1], 1.0)
    lab8 = dl[:, 1:9] * di
    h1 = jnp.tanh(xws_ref[...] + s1 * di + dot(lab8, wbl_ref[...])
                  + bb_ref[...])
    h1n_ref[...] = dot(h1, wdn_ref[...])
    pre2_ref[...] = (dot(h1, wds_ref[...]) + dot(lab8, wdl_ref[...])
                     + bd_ref[...])
    dinv_ref[...] = jnp.broadcast_to(di, (RB, 8))


def _tc2(pre2_ref, s2_ref, dinv_ref, wa_ref, wb_ref, z_ref, ab_ref):
    dot = functools.partial(jnp.dot, preferred_element_type=_f32,
                            precision=lax.Precision.HIGHEST)
    s2p = s2_ref[...]
    z = jnp.tanh(pre2_ref[...] + (s2p[0] + s2p[1]) * dinv_ref[:, 0:1])
    z_ref[...] = z
    ab_ref[...] = jnp.concatenate(
        [dot(z, wa_ref[...]), dot(z, wb_ref[...])], axis=1)


def _sc_loss():
    scratch = (
        [pltpu.VMEM((N * 8,), _f32)]
        + [pltpu.VMEM((LB,), jnp.int32) for _ in range(5)]
        + [pltpu.VMEM((LB,), _f32) for _ in range(4)]
    )

    @functools.partial(
        pl.kernel,
        out_type=jax.ShapeDtypeStruct((4, NLP), _f32),
        mesh=_mesh(),
        scratch_types=scratch,
        compiler_params=pltpu.CompilerParams(use_tc_tiling_on_sc=False, needs_layout_passes=False))
    def k(ab_h, s0_h, s1_h, su_h, yl_h, yn_h, out_h,
          ab_v, s0_v, s1_v, su_v, yl_v, yn_v, ul_v, tl_v, un_v, tn_v):
        cid = lax.axis_index("c")
        sid = lax.axis_index("s")
        wid = sid * NC + cid
        pltpu.sync_copy(ab_h, ab_v)

        def blk(b, carry):
            base = wid * Q + b * LB
            win = pl.ds(base, LB)
            pltpu.sync_copy(s0_h.at[win], s0_v)
            pltpu.sync_copy(s1_h.at[win], s1_v)
            pltpu.sync_copy(su_h.at[win], su_v)
            pltpu.sync_copy(yl_h.at[win], yl_v)
            pltpu.sync_copy(yn_h.at[win], yn_v)

            def ch(i, c2):
                sl = pl.ds(i * 16, 16)
                i0 = s0_v[sl] * 8
                i1 = s1_v[sl] * 8 + 4
                i2 = su_v[sl] * 8 + 4
                a0 = plsc.load_gather(ab_v, [i0])
                a1 = plsc.load_gather(ab_v, [i0 + 1])
                a2 = plsc.load_gather(ab_v, [i0 + 2])
                b0 = plsc.load_gather(ab_v, [i1])
                b1 = plsc.load_gather(ab_v, [i1 + 1])
                b2 = plsc.load_gather(ab_v, [i1 + 2])
                c0 = plsc.load_gather(ab_v, [i2])
                c1 = plsc.load_gather(ab_v, [i2 + 1])
                c2g = plsc.load_gather(ab_v, [i2 + 2])

                def softstats(p0, p1, p2, yv):
                    m = jnp.maximum(p0, jnp.maximum(p1, p2))
                    u = (jnp.exp(p0 - m) + jnp.exp(p1 - m)
                         + jnp.exp(p2 - m))
                    py = jnp.where(yv == 0, p0, jnp.where(yv == 1, p1, p2))
                    return u, m - py

                ul, tl = softstats(a0 + b0, a1 + b1, a2 + b2, yl_v[sl])
                un, tn = softstats(a0 + c0, a1 + c1, a2 + c2g, yn_v[sl])
                ul_v[sl] = ul
                tl_v[sl] = tl
                un_v[sl] = un
                tn_v[sl] = tn
                return c2

            lax.fori_loop(0, NCH, ch, 0)
            pltpu.sync_copy(ul_v, out_h.at[0, win])
            pltpu.sync_copy(tl_v, out_h.at[1, win])
            pltpu.sync_copy(un_v, out_h.at[2, win])
            pltpu.sync_copy(tn_v, out_h.at[3, win])
            return carry

        lax.fori_loop(0, NBLK, blk, 0)

    return k


LCB = 6400
LGRID = NLP // LCB


def _tc3(u_ref, out_ref):
    i = pl.program_id(0)
    x = u_ref[...]
    c = jnp.log(x[0:1]) + x[1:2] + jnp.log(x[2:3]) + x[3:4]
    col = lax.broadcasted_iota(jnp.int32, (1, LCB), 1) + i * LCB
    s = (jnp.sum(jnp.where(col < NL, c, 0.0), axis=1, keepdims=True)
         * (1.0 / (2 * NL)))

    @pl.when(i == 0)
    def _():
        out_ref[...] = jnp.zeros_like(out_ref)

    out_ref[...] += s


def _pad1(a, n, val=0):
    return jnp.concatenate([a, jnp.full((n - a.shape[0],), val, a.dtype)])


def kernel(X, W_base, b_base, W_deep, b_deep, reg_W, train_edges, y_train,
           sorted_train_edges, surrogates, y):
    wbs, wbn, wbl = W_base[:DF], W_base[DF:2 * DF], W_base[2 * DF:]
    wds, wdn, wdl = W_deep[:H], W_deep[H:2 * H], W_deep[2 * H:]
    wbl8 = jnp.zeros((8, H), _f32).at[:NLAB].set(wbl)
    wdl8 = jnp.zeros((8, H), _f32).at[:NLAB].set(wdl)
    wa4 = jnp.zeros((H, 4), _f32).at[:, :NLAB].set(reg_W[:H])
    wb4 = jnp.zeros((H, 4), _f32).at[:, :NLAB].set(reg_W[H:])
    bb = b_base.reshape(1, H)
    bd = b_deep.reshape(1, H)

    src = _pad1(train_edges[:, 0], E_PAD).reshape(EROWS, CHUNK)
    dst = _pad1(train_edges[:, 1], E_PAD, DUMMY).reshape(EROWS, CHUNK)
    ytr = _pad1(y_train, E_PAD).reshape(EROWS, CHUNK)

    ltab_np = np.zeros((8, 16), np.float32)
    ltab_np[:NLAB, 0] = 1.0
    ltab_np[np.arange(NLAB), 1 + np.arange(NLAB)] = 1.0
    ltab = jnp.asarray(ltab_np)

    s0 = _pad1(sorted_train_edges[:, :, 0].reshape(-1), NLP)
    s1 = _pad1(sorted_train_edges[:, :, 1].reshape(-1), NLP)
    su = _pad1(surrogates.reshape(-1), NLP)
    yl = _pad1(y[:NL], NLP)
    yn = _pad1(y[NL:], NLP)

    z64 = jnp.zeros((RPT, H), _f32)
    z16 = jnp.zeros((RPT, 16), _f32)

    xws, xwn = pl.pallas_call(
        _tc0,
        grid=(GRID,),
        in_specs=[pl.BlockSpec((RB, DF), lambda i: (i, 0)),
                  pl.BlockSpec((DF, H), lambda i: (0, 0)),
                  pl.BlockSpec((DF, H), lambda i: (0, 0))],
        out_specs=[pl.BlockSpec((RB, H), lambda i: (i, 0))] * 2,
        out_shape=[jax.ShapeDtypeStruct((N, H), _f32)] * 2,
    )(X, wbs, wbn)

    s1p, dlp = _sc_edges(True)(xwn, ltab, src, dst, ytr, z64, z16)

    h1n, pre2, dinv = pl.pallas_call(
        _tc1,
        grid=(GRID,),
        in_specs=[pl.BlockSpec((RB, H), lambda i: (i, 0)),
                  pl.BlockSpec((NC, RB, H), lambda i: (0, i, 0)),
                  pl.BlockSpec((NC, RB, 16), lambda i: (0, i, 0)),
                  pl.BlockSpec((8, H), lambda i: (0, 0)),
                  pl.BlockSpec((1, H), lambda i: (0, 0)),
                  pl.BlockSpec((H, H), lambda i: (0, 0)),
                  pl.BlockSpec((H, H), lambda i: (0, 0)),
                  pl.BlockSpec((8, H), lambda i: (0, 0)),
                  pl.BlockSpec((1, H), lambda i: (0, 0))],
        out_specs=[pl.BlockSpec((RB, H), lambda i: (i, 0)),
                   pl.BlockSpec((RB, H), lambda i: (i, 0)),
                   pl.BlockSpec((RB, 8), lambda i: (i, 0))],
        out_shape=[jax.ShapeDtypeStruct((N, H), _f32),
                   jax.ShapeDtypeStruct((N, H), _f32),
                   jax.ShapeDtypeStruct((N, 8), _f32)],
    )(xws, s1p, dlp, wbl8, bb, wds, wdn, wdl8, bd)

    (s2p,) = _sc_edges(False)(h1n, src, dst, z64)

    z, ab = pl.pallas_call(
        _tc2,
        grid=(GRID,),
        in_specs=[pl.BlockSpec((RB, H), lambda i: (i, 0)),
                  pl.BlockSpec((NC, RB, H), lambda i: (0, i, 0)),
                  pl.BlockSpec((RB, 8), lambda i: (i, 0)),
                  pl.BlockSpec((H, 4), lambda i: (0, 0)),
                  pl.BlockSpec((H, 4), lambda i: (0, 0))],
        out_specs=[pl.BlockSpec((RB, H), lambda i: (i, 0)),
                   pl.BlockSpec((RB, 8), lambda i: (i, 0))],
        out_shape=[jax.ShapeDtypeStruct((N, H), _f32),
                   jax.ShapeDtypeStruct((N, 8), _f32)],
    )(pre2, s2p, dinv, wa4, wb4)

    u4 = _sc_loss()(ab.reshape(N * 8), s0, s1, su, yl, yn)

    tot = pl.pallas_call(
        _tc3,
        grid=(LGRID,),
        in_specs=[pl.BlockSpec((4, LCB), lambda i: (0, i))],
        out_specs=pl.BlockSpec((1, 1), lambda i: (0, 0)),
        out_shape=jax.ShapeDtypeStruct((1, 1), _f32),
    )(u4)

    return tot[0, 0], z


# speedup vs baseline: 3.1656x; 1.0189x over previous
import functools

import jax
import jax.numpy as jnp
import numpy as np
from jax import lax
from jax.experimental import pallas as pl
from jax.experimental.pallas import tpu as pltpu
from jax.experimental.pallas import tpu_sc as plsc

N = 10000
DF = 128
NLAB = 3
H = 64
E = 320000
EP = 100000
NL = NLAB * EP

NC, NS = 2, 16
NW = NC * NS

NP = 10112
RPT = NP // NS
DUMMY = N

CHUNK = 128
CPT = 80
EPT = CPT * CHUNK
E_PAD = NW * EPT
EROWS = E_PAD // CHUNK

Q = 9600
NLP = NW * Q
LB = 640
NBLK = Q // LB
NCH = LB // 16

RB = 400
GRID = N // RB

_f32 = jnp.float32


def _mesh():
    return plsc.VectorSubcoreMesh(core_axis_name="c", subcore_axis_name="s",
                                  num_cores=NC, num_subcores=NS)


def _tc0(x_ref, ws_ref, wn_ref, xws_ref, xwn_ref):
    x = x_ref[...]
    dot = functools.partial(jnp.dot, preferred_element_type=_f32,
                            precision=lax.Precision.HIGHEST)
    xws_ref[...] = dot(x, ws_ref[...])
    xwn_ref[...] = dot(x, wn_ref[...])


def _sc_edges(with_lab, K):
    SB = CPT // K
    HALF = SB // 2
    scratch = [
        pltpu.VMEM((CPT, CHUNK), jnp.int32),
        pltpu.VMEM((CPT, CHUNK), jnp.int32),
        pltpu.VMEM((K * CHUNK, H), _f32),
        pltpu.VMEM((K * CHUNK, H), _f32),
        pltpu.VMEM_SHARED((NP, H), _f32),
        pltpu.SemaphoreType.DMA,
        pltpu.SemaphoreType.DMA,
        pltpu.SemaphoreType.DMA,
        pltpu.SemaphoreType.DMA,
    ]
    out = [jax.ShapeDtypeStruct((NC, NP, H), _f32)]
    if with_lab:
        scratch += [
            pltpu.VMEM((CPT, CHUNK), jnp.int32),
            pltpu.VMEM((K * CHUNK, 16), _f32),
            pltpu.VMEM((K * CHUNK, 16), _f32),
            pltpu.VMEM_SHARED((NP, 16), _f32),
        ]
        out += [jax.ShapeDtypeStruct((NC, NP, 16), _f32)]

    @functools.partial(pl.kernel, out_type=out, mesh=_mesh(),
                       scratch_types=scratch,
                       compiler_params=pltpu.CompilerParams(
                           use_tc_tiling_on_sc=False, needs_layout_passes=False))
    def k(*refs):
        if with_lab:
            (tab_h, ltab_h, src_h, dst_h, y_h, z64_h, z16_h,
             s_h, dl_h, src_v, dst_v, rbufA, rbufB, acc,
             gsemA, gsemB, ssemA, ssemB,
             y_v, lbufA, lbufB, acc16) = refs
        else:
            (tab_h, src_h, dst_h, z64_h,
             s_h, src_v, dst_v, rbufA, rbufB, acc,
             gsemA, gsemB, ssemA, ssemB) = refs
            y_v = lbufA = lbufB = acc16 = None
        cid = lax.axis_index("c")
        sid = lax.axis_index("s")
        wid = sid * NC + cid
        rows = pl.ds(sid * RPT, RPT)
        pltpu.sync_copy(z64_h, acc.at[rows])
        if with_lab:
            pltpu.sync_copy(z16_h, acc16.at[rows])
        erows = pl.ds(wid * CPT, CPT)
        pltpu.sync_copy(src_h.at[erows], src_v)
        pltpu.sync_copy(dst_h.at[erows], dst_v)
        if with_lab:
            pltpu.sync_copy(y_h.at[erows], y_v)
        plsc.subcore_barrier()

        def chunk_slices(rbuf, lbuf, kk):
            sl = pl.ds(kk * CHUNK, CHUNK)
            return rbuf.at[sl], (lbuf.at[sl] if with_lab else None)

        def fire_g(blk, rbuf, lbuf, sem):
            for kk in range(K):
                j = blk * K + kk
                rsl, lsl = chunk_slices(rbuf, lbuf, kk)
                pltpu.async_copy(tab_h.at[src_v.at[j]], rsl, sem)
                if with_lab:
                    pltpu.async_copy(ltab_h.at[y_v.at[j]], lsl, sem)

        def wait_g(rbuf, lbuf, sem):
            for kk in range(K):
                rsl, lsl = chunk_slices(rbuf, lbuf, kk)
                pltpu.make_async_copy(tab_h.at[src_v.at[0]], rsl, sem).wait()
                if with_lab:
                    pltpu.make_async_copy(ltab_h.at[y_v.at[0]], lsl,
                                          sem).wait()

        def fire_s(blk, rbuf, lbuf, sem):
            for kk in range(K):
                j = blk * K + kk
                rsl, lsl = chunk_slices(rbuf, lbuf, kk)
                pltpu.async_copy(rsl, acc.at[dst_v.at[j]], sem, add=True)
                if with_lab:
                    pltpu.async_copy(lsl, acc16.at[dst_v.at[j]], sem,
                                     add=True)

        def wait_s(rbuf, lbuf, sem):
            for kk in range(K):
                rsl, lsl = chunk_slices(rbuf, lbuf, kk)
                pltpu.make_async_copy(rsl, acc.at[dst_v.at[0]], sem).wait()
                if with_lab:
                    pltpu.make_async_copy(lsl, acc16.at[dst_v.at[0]],
                                          sem).wait()

        fire_g(0, rbufA, lbufA, gsemA)

        def body(i, carry):
            @pl.when(i > 0)
            def _():
                wait_s(rbufB, lbufB, ssemB)

            fire_g(2 * i + 1, rbufB, lbufB, gsemB)
            wait_g(rbufA, lbufA, gsemA)
            fire_s(2 * i, rbufA, lbufA, ssemA)
            wait_g(rbufB, lbufB, gsemB)
            wait_s(rbufA, lbufA, ssemA)

            @pl.when(i < HALF - 1)
            def _():
                fire_g(2 * i + 2, rbufA, lbufA, gsemA)

            fire_s(2 * i + 1, rbufB, lbufB, ssemB)
            return carry

        lax.fori_loop(0, HALF, body, 0)
        wait_s(rbufB, lbufB, ssemB)
        plsc.subcore_barrier()
        pltpu.sync_copy(acc.at[rows], s_h.at[cid, rows])
        if with_lab:
            pltpu.sync_copy(acc16.at[rows], dl_h.at[cid, rows])

    return k


def _tc1(xws_ref, s1_ref, dl_ref, wbl_ref, bb_ref, wds_ref, wdn_ref,
         wdl_ref, bd_ref, h1n_ref, pre2_ref, dinv_ref):
    dot = functools.partial(jnp.dot, preferred_element_type=_f32,
                            precision=lax.Precision.HIGHEST)
    s1p = s1_ref[...]
    dlp = dl_ref[...]
    s1 = s1p[0] + s1p[1]
    dl = dlp[0] + dlp[1]
    di = 1.0 / jnp.maximum(dl[:, 0:1], 1.0)
    lab8 = dl[:, 1:9] * di
    h1 = jnp.tanh(xws_ref[...] + s1 * di + dot(lab8, wbl_ref[...])
                  + bb_ref[...])
    h1n_ref[...] = dot(h1, wdn_ref[...])
    pre2_ref[...] = (dot(h1, wds_ref[...]) + dot(lab8, wdl_ref[...])
                     + bd_ref[...])
    dinv_ref[...] = jnp.broadcast_to(di, (RB, 8))


def _tc2(pre2_ref, s2_ref, dinv_ref, wa_ref, wb_ref, z_ref, ab_ref):
    dot = functools.partial(jnp.dot, preferred_element_type=_f32,
                            precision=lax.Precision.HIGHEST)
    s2p = s2_ref[...]
    z = jnp.tanh(pre2_ref[...] + (s2p[0] + s2p[1]) * dinv_ref[:, 0:1])
    z_ref[...] = z
    ab_ref[...] = jnp.concatenate(
        [dot(z, wa_ref[...]), dot(z, wb_ref[...])], axis=1)


def _sc_loss():
    scratch = (
        [pltpu.VMEM((N * 8,), _f32)]
        + [pltpu.VMEM((LB,), jnp.int32) for _ in range(5)]
        + [pltpu.VMEM((LB,), _f32) for _ in range(4)]
    )

    @functools.partial(
        pl.kernel,
        out_type=jax.ShapeDtypeStruct((4, NLP), _f32),
        mesh=_mesh(),
        scratch_types=scratch,
        compiler_params=pltpu.CompilerParams(use_tc_tiling_on_sc=False, needs_layout_passes=False))
    def k(ab_h, s0_h, s1_h, su_h, yl_h, yn_h, out_h,
          ab_v, s0_v, s1_v, su_v, yl_v, yn_v, ul_v, tl_v, un_v, tn_v):
        cid = lax.axis_index("c")
        sid = lax.axis_index("s")
        wid = sid * NC + cid
        pltpu.sync_copy(ab_h, ab_v)

        def blk(b, carry):
            base = wid * Q + b * LB
            win = pl.ds(base, LB)
            pltpu.sync_copy(s0_h.at[win], s0_v)
            pltpu.sync_copy(s1_h.at[win], s1_v)
            pltpu.sync_copy(su_h.at[win], su_v)
            pltpu.sync_copy(yl_h.at[win], yl_v)
            pltpu.sync_copy(yn_h.at[win], yn_v)

            def ch(i, c2):
                sl = pl.ds(i * 16, 16)
                i0 = s0_v[sl] * 8
                i1 = s1_v[sl] * 8 + 4
                i2 = su_v[sl] * 8 + 4
                a0 = plsc.load_gather(ab_v, [i0])
                a1 = plsc.load_gather(ab_v, [i0 + 1])
                a2 = plsc.load_gather(ab_v, [i0 + 2])
                b0 = plsc.load_gather(ab_v, [i1])
                b1 = plsc.load_gather(ab_v, [i1 + 1])
                b2 = plsc.load_gather(ab_v, [i1 + 2])
                c0 = plsc.load_gather(ab_v, [i2])
                c1 = plsc.load_gather(ab_v, [i2 + 1])
                c2g = plsc.load_gather(ab_v, [i2 + 2])

                def softstats(p0, p1, p2, yv):
                    m = jnp.maximum(p0, jnp.maximum(p1, p2))
                    u = (jnp.exp(p0 - m) + jnp.exp(p1 - m)
                         + jnp.exp(p2 - m))
                    py = jnp.where(yv == 0, p0, jnp.where(yv == 1, p1, p2))
                    return u, m - py

                ul, tl = softstats(a0 + b0, a1 + b1, a2 + b2, yl_v[sl])
                un, tn = softstats(a0 + c0, a1 + c1, a2 + c2g, yn_v[sl])
                ul_v[sl] = ul
                tl_v[sl] = tl
                un_v[sl] = un
                tn_v[sl] = tn
                return c2

            lax.fori_loop(0, NCH, ch, 0)
            pltpu.sync_copy(ul_v, out_h.at[0, win])
            pltpu.sync_copy(tl_v, out_h.at[1, win])
            pltpu.sync_copy(un_v, out_h.at[2, win])
            pltpu.sync_copy(tn_v, out_h.at[3, win])
            return carry

        lax.fori_loop(0, NBLK, blk, 0)

    return k


LCB = 6400
LGRID = NLP // LCB


def _tc3(u_ref, out_ref):
    i = pl.program_id(0)
    x = u_ref[...]
    c = jnp.log(x[0:1]) + x[1:2] + jnp.log(x[2:3]) + x[3:4]
    col = lax.broadcasted_iota(jnp.int32, (1, LCB), 1) + i * LCB
    s = (jnp.sum(jnp.where(col < NL, c, 0.0), axis=1, keepdims=True)
         * (1.0 / (2 * NL)))

    @pl.when(i == 0)
    def _():
        out_ref[...] = jnp.zeros_like(out_ref)

    out_ref[...] += s


def _pad1(a, n, val=0):
    return jnp.concatenate([a, jnp.full((n - a.shape[0],), val, a.dtype)])


def kernel(X, W_base, b_base, W_deep, b_deep, reg_W, train_edges, y_train,
           sorted_train_edges, surrogates, y):
    wbs, wbn, wbl = W_base[:DF], W_base[DF:2 * DF], W_base[2 * DF:]
    wds, wdn, wdl = W_deep[:H], W_deep[H:2 * H], W_deep[2 * H:]
    wbl8 = jnp.zeros((8, H), _f32).at[:NLAB].set(wbl)
    wdl8 = jnp.zeros((8, H), _f32).at[:NLAB].set(wdl)
    wa4 = jnp.zeros((H, 4), _f32).at[:, :NLAB].set(reg_W[:H])
    wb4 = jnp.zeros((H, 4), _f32).at[:, :NLAB].set(reg_W[H:])
    bb = b_base.reshape(1, H)
    bd = b_deep.reshape(1, H)

    src = _pad1(train_edges[:, 0], E_PAD).reshape(EROWS, CHUNK)
    dst = _pad1(train_edges[:, 1], E_PAD, DUMMY).reshape(EROWS, CHUNK)
    ytr = _pad1(y_train, E_PAD).reshape(EROWS, CHUNK)

    ltab_np = np.zeros((8, 16), np.float32)
    ltab_np[:NLAB, 0] = 1.0
    ltab_np[np.arange(NLAB), 1 + np.arange(NLAB)] = 1.0
    ltab = jnp.asarray(ltab_np)

    s0 = _pad1(sorted_train_edges[:, :, 0].reshape(-1), NLP)
    s1 = _pad1(sorted_train_edges[:, :, 1].reshape(-1), NLP)
    su = _pad1(surrogates.reshape(-1), NLP)
    yl = _pad1(y[:NL], NLP)
    yn = _pad1(y[NL:], NLP)

    z64 = jnp.zeros((RPT, H), _f32)
    z16 = jnp.zeros((RPT, 16), _f32)

    xws, xwn = pl.pallas_call(
        _tc0,
        grid=(GRID,),
        in_specs=[pl.BlockSpec((RB, DF), lambda i: (i, 0)),
                  pl.BlockSpec((DF, H), lambda i: (0, 0)),
                  pl.BlockSpec((DF, H), lambda i: (0, 0))],
        out_specs=[pl.BlockSpec((RB, H), lambda i: (i, 0))] * 2,
        out_shape=[jax.ShapeDtypeStruct((N, H), _f32)] * 2,
    )(X, wbs, wbn)

    s1p, dlp = _sc_edges(True, 2)(xwn, ltab, src, dst, ytr, z64, z16)

    h1n, pre2, dinv = pl.pallas_call(
        _tc1,
        grid=(GRID,),
        in_specs=[pl.BlockSpec((RB, H), lambda i: (i, 0)),
                  pl.BlockSpec((NC, RB, H), lambda i: (0, i, 0)),
                  pl.BlockSpec((NC, RB, 16), lambda i: (0, i, 0)),
                  pl.BlockSpec((8, H), lambda i: (0, 0)),
                  pl.BlockSpec((1, H), lambda i: (0, 0)),
                  pl.BlockSpec((H, H), lambda i: (0, 0)),
                  pl.BlockSpec((H, H), lambda i: (0, 0)),
                  pl.BlockSpec((8, H), lambda i: (0, 0)),
                  pl.BlockSpec((1, H), lambda i: (0, 0))],
        out_specs=[pl.BlockSpec((RB, H), lambda i: (i, 0)),
                   pl.BlockSpec((RB, H), lambda i: (i, 0)),
                   pl.BlockSpec((RB, 8), lambda i: (i, 0))],
        out_shape=[jax.ShapeDtypeStruct((N, H), _f32),
                   jax.ShapeDtypeStruct((N, H), _f32),
                   jax.ShapeDtypeStruct((N, 8), _f32)],
    )(xws, s1p, dlp, wbl8, bb, wds, wdn, wdl8, bd)

    (s2p,) = _sc_edges(False, 4)(h1n, src, dst, z64)

    z, ab = pl.pallas_call(
        _tc2,
        grid=(GRID,),
        in_specs=[pl.BlockSpec((RB, H), lambda i: (i, 0)),
                  pl.BlockSpec((NC, RB, H), lambda i: (0, i, 0)),
                  pl.BlockSpec((RB, 8), lambda i: (i, 0)),
                  pl.BlockSpec((H, 4), lambda i: (0, 0)),
                  pl.BlockSpec((H, 4), lambda i: (0, 0))],
        out_specs=[pl.BlockSpec((RB, H), lambda i: (i, 0)),
                   pl.BlockSpec((RB, 8), lambda i: (i, 0))],
        out_shape=[jax.ShapeDtypeStruct((N, H), _f32),
                   jax.ShapeDtypeStruct((N, 8), _f32)],
    )(pre2, s2p, dinv, wa4, wb4)

    u4 = _sc_loss()(ab.reshape(N * 8), s0, s1, su, yl, yn)

    tot = pl.pallas_call(
        _tc3,
        grid=(LGRID,),
        in_specs=[pl.BlockSpec((4, LCB), lambda i: (0, i))],
        out_specs=pl.BlockSpec((1, 1), lambda i: (0, 0)),
        out_shape=jax.ShapeDtypeStruct((1, 1), _f32),
    )(u4)

    return tot[0, 0], z


# speedup vs baseline: 24.4323x; 7.7180x over previous
import functools

import jax
import jax.numpy as jnp
import numpy as np
from jax import lax
from jax.experimental import pallas as pl
from jax.experimental.pallas import tpu as pltpu
from jax.experimental.pallas import tpu_sc as plsc

N = 10000
DF = 128
NLAB = 3
H = 64
E = 320000
EP = 100000
NL = NLAB * EP

NC, NS = 2, 16
NW = NC * NS

NP = 10112
RPT = NP // NS
DUMMY = N

CHUNK = 128
CPT = 80
EPT = CPT * CHUNK
E_PAD = NW * EPT
EROWS = E_PAD // CHUNK

Q = 9600
NLP = NW * Q
LB = 800
NBLK = Q // LB
NCH = LB // 16
PAIRS = NBLK // 2

RB = 2000
GRID = N // RB

_f32 = jnp.float32

_LOG_COEF = (0.6931471805599456, 0.5000009054167627, -0.12500041841408105,
             0.04164875283632623, -0.015616722596026157,
             0.006347559226599455, -0.00264923158324848,
             0.0009052258360685269, -0.00039094821307566503,
             0.00040359603284204326, -0.00018366676407698503)


def _mesh():
    return plsc.VectorSubcoreMesh(core_axis_name="c", subcore_axis_name="s",
                                  num_cores=NC, num_subcores=NS)


_SC_PARAMS = pltpu.CompilerParams(use_tc_tiling_on_sc=False,
                                  needs_layout_passes=False)


def _tc0(x_ref, wn_ref, xwn_ref):
    xwn_ref[...] = jnp.dot(x_ref[...], wn_ref[...],
                           preferred_element_type=_f32)


def _sc_edges(with_lab, K):
    SBLK = CPT // K
    HLF = SBLK // 2
    scratch = [
        pltpu.VMEM((CPT, CHUNK), jnp.int32),
        pltpu.VMEM((CPT, CHUNK), jnp.int32),
        pltpu.VMEM((K * CHUNK, H), _f32),
        pltpu.VMEM((K * CHUNK, H), _f32),
        pltpu.VMEM_SHARED((NP, H), _f32),
        pltpu.SemaphoreType.DMA,
        pltpu.SemaphoreType.DMA,
        pltpu.SemaphoreType.DMA,
        pltpu.SemaphoreType.DMA,
    ]
    out = [jax.ShapeDtypeStruct((NP, 2 * H), _f32)]
    if with_lab:
        scratch += [
            pltpu.VMEM((CPT, CHUNK), jnp.int32),
            pltpu.VMEM((K * CHUNK, 16), _f32),
            pltpu.VMEM((K * CHUNK, 16), _f32),
            pltpu.VMEM_SHARED((NP, 16), _f32),
        ]
        out += [jax.ShapeDtypeStruct((NP, 2 * H), _f32)]

    @functools.partial(pl.kernel, out_type=out, mesh=_mesh(),
                       scratch_types=scratch, compiler_params=_SC_PARAMS)
    def k(*refs):
        if with_lab:
            (tab_h, ltab_h, src_h, dst_h, y_h, z64_h, z16_h,
             s_h, dl_h, src_v, dst_v, rbufA, rbufB, acc,
             gsemA, gsemB, ssemA, ssemB,
             y_v, lbufA, lbufB, acc16) = refs
        else:
            (tab_h, src_h, dst_h, z64_h,
             s_h, src_v, dst_v, rbufA, rbufB, acc,
             gsemA, gsemB, ssemA, ssemB) = refs
            y_v = lbufA = lbufB = acc16 = None
        cid = lax.axis_index("c")
        sid = lax.axis_index("s")
        wid = sid * NC + cid
        rows = pl.ds(sid * RPT, RPT)
        pltpu.sync_copy(z64_h, acc.at[rows])
        if with_lab:
            pltpu.sync_copy(z16_h, acc16.at[rows])
        erows = pl.ds(wid * CPT, CPT)
        pltpu.sync_copy(src_h.at[erows], src_v)
        pltpu.sync_copy(dst_h.at[erows], dst_v)
        if with_lab:
            pltpu.sync_copy(y_h.at[erows], y_v)
        plsc.subcore_barrier()

        def chunk_slices(rbuf, lbuf, kk):
            sl = pl.ds(kk * CHUNK, CHUNK)
            return rbuf.at[sl], (lbuf.at[sl] if with_lab else None)

        def fire_g(blk, rbuf, lbuf, sem):
            for kk in range(K):
                j = blk * K + kk
                rsl, lsl = chunk_slices(rbuf, lbuf, kk)
                pltpu.async_copy(tab_h.at[src_v.at[j]], rsl, sem)
                if with_lab:
                    pltpu.async_copy(ltab_h.at[y_v.at[j]], lsl, sem)

        def wait_g(rbuf, lbuf, sem):
            for kk in range(K):
                rsl, lsl = chunk_slices(rbuf, lbuf, kk)
                pltpu.make_async_copy(tab_h.at[src_v.at[0]], rsl, sem).wait()
                if with_lab:
                    pltpu.make_async_copy(ltab_h.at[y_v.at[0]], lsl,
                                          sem).wait()

        def fire_s(blk, rbuf, lbuf, sem):
            for kk in range(K):
                j = blk * K + kk
                rsl, lsl = chunk_slices(rbuf, lbuf, kk)
                pltpu.async_copy(rsl, acc.at[dst_v.at[j]], sem, add=True)
                if with_lab:
                    pltpu.async_copy(lsl, acc16.at[dst_v.at[j]], sem,
                                     add=True)

        def wait_s(rbuf, lbuf, sem):
            for kk in range(K):
                rsl, lsl = chunk_slices(rbuf, lbuf, kk)
                pltpu.make_async_copy(rsl, acc.at[dst_v.at[0]], sem).wait()
                if with_lab:
                    pltpu.make_async_copy(lsl, acc16.at[dst_v.at[0]],
                                          sem).wait()

        fire_g(0, rbufA, lbufA, gsemA)

        def body(i, carry):
            @pl.when(i > 0)
            def _():
                wait_s(rbufB, lbufB, ssemB)

            fire_g(2 * i + 1, rbufB, lbufB, gsemB)
            wait_g(rbufA, lbufA, gsemA)
            fire_s(2 * i, rbufA, lbufA, ssemA)
            wait_g(rbufB, lbufB, gsemB)
            wait_s(rbufA, lbufA, ssemA)

            @pl.when(i < HLF - 1)
            def _():
                fire_g(2 * i + 2, rbufA, lbufA, gsemA)

            fire_s(2 * i + 1, rbufB, lbufB, ssemB)
            return carry

        lax.fori_loop(0, HLF, body, 0)
        wait_s(rbufB, lbufB, ssemB)
        plsc.subcore_barrier()
        pltpu.sync_copy(acc.at[rows], s_h.at[rows, pl.ds(cid * H, H)])
        if with_lab:
            pltpu.sync_copy(acc16.at[rows],
                            dl_h.at[rows, pl.ds(cid * 16, 16)])

    return k


def _tc1(x_ref, wbs_ref, s1_ref, dl_ref, wbl_ref, bb_ref, wds_ref, wdn_ref,
         wdl_ref, bd_ref, h1n_ref, pre2_ref, dinv_ref):
    dot = functools.partial(jnp.dot, preferred_element_type=_f32)
    s1p = s1_ref[...]
    dlp = dl_ref[...]
    s1 = s1p[:, :H] + s1p[:, H:]
    dl = dlp[:, :16] + dlp[:, 16:32]
    di = 1.0 / jnp.maximum(dl[:, 0:1], 1.0)
    lab8 = dl[:, 1:9] * di
    h1 = jnp.tanh(dot(x_ref[...], wbs_ref[...]) + s1 * di
                  + dot(lab8, wbl_ref[...]) + bb_ref[...])
    h1n_ref[...] = dot(h1, wdn_ref[...])
    pre2_ref[...] = (dot(h1, wds_ref[...]) + dot(lab8, wdl_ref[...])
                     + bd_ref[...])
    dinv_ref[...] = jnp.broadcast_to(di, (RB, 8))


def _tc2(pre2_ref, s2_ref, dinv_ref, wa_ref, wb_ref, z_ref, ab_ref):
    dot = functools.partial(jnp.dot, preferred_element_type=_f32)
    s2p = s2_ref[...]
    s2 = s2p[:, :H] + s2p[:, H:]
    z = jnp.tanh(pre2_ref[...] + s2 * dinv_ref[:, 0:1])
    z_ref[...] = z
    ab_ref[...] = jnp.concatenate(
        [dot(z, wa_ref[...]), dot(z, wb_ref[...])], axis=1)


def _logp(u):
    w = u - 2.0
    acc = jnp.full((16,), _LOG_COEF[-1], _f32)
    for c in _LOG_COEF[-2::-1]:
        acc = acc * w + c
    return acc


def _sc_loss():
    scratch = (
        [pltpu.VMEM((N, 8), _f32)]
        + [pltpu.VMEM((5, LB), jnp.int32) for _ in range(2)]
        + [pltpu.VMEM((16,), _f32)]
        + [pltpu.SemaphoreType.DMA for _ in range(2)]
    )

    @functools.partial(
        pl.kernel,
        out_type=jax.ShapeDtypeStruct((NW, CHUNK), _f32),
        mesh=_mesh(),
        scratch_types=scratch,
        compiler_params=_SC_PARAMS)
    def k(ab_h, s0_h, s1_h, su_h, yl_h, yn_h, out_h,
          ab_v, inA, inB, rsum_v, isemA, isemB):
        cid = lax.axis_index("c")
        sid = lax.axis_index("s")
        wid = sid * NC + cid
        pltpu.sync_copy(ab_h, ab_v)
        idx_hs = (s0_h, s1_h, su_h, yl_h, yn_h)

        def stage_in(b, ibuf, sem):
            win = pl.ds(wid * Q + b * LB, LB)
            for r, hh in enumerate(idx_hs):
                pltpu.async_copy(hh.at[win], ibuf.at[r], sem)

        def wait_in(ibuf, sem):
            for r, hh in enumerate(idx_hs):
                pltpu.make_async_copy(hh.at[pl.ds(0, LB)], ibuf.at[r],
                                      sem).wait()

        czero = jnp.zeros((16,), jnp.int32)

        def compute(ibuf, blk, total):
            base = wid * Q + blk * LB

            def ch(i, tot):
                sl = pl.ds(i * 16, 16)
                i0 = ibuf[0, sl]
                i1 = ibuf[1, sl]
                i2 = ibuf[2, sl]
                a0 = plsc.load_gather(ab_v, [i0, czero])
                a1 = plsc.load_gather(ab_v, [i0, czero + 1])
                a2 = plsc.load_gather(ab_v, [i0, czero + 2])
                b0 = plsc.load_gather(ab_v, [i1, czero + 4])
                b1 = plsc.load_gather(ab_v, [i1, czero + 5])
                b2 = plsc.load_gather(ab_v, [i1, czero + 6])
                c0 = plsc.load_gather(ab_v, [i2, czero + 4])
                c1 = plsc.load_gather(ab_v, [i2, czero + 5])
                c2g = plsc.load_gather(ab_v, [i2, czero + 6])

                def nll(p0, p1, p2, yv):
                    m = jnp.maximum(p0, jnp.maximum(p1, p2))
                    u = (jnp.exp(p0 - m) + jnp.exp(p1 - m)
                         + jnp.exp(p2 - m))
                    py = jnp.where(yv == 0, p0, jnp.where(yv == 1, p1, p2))
                    return _logp(u) + m - py

                c_l = nll(a0 + b0, a1 + b1, a2 + b2, ibuf[3, sl])
                c_n = nll(a0 + c0, a1 + c1, a2 + c2g, ibuf[4, sl])
                gidx = base + i * 16 + lax.iota(jnp.int32, 16)
                return tot + jnp.where(gidx < NL, c_l + c_n, 0.0)

            return lax.fori_loop(0, NCH, ch, total)

        stage_in(0, inA, isemA)
        stage_in(1, inB, isemB)

        def body(p, total):
            wait_in(inA, isemA)
            total = compute(inA, 2 * p, total)

            @pl.when(p < PAIRS - 1)
            def _():
                stage_in(2 * p + 2, inA, isemA)

            wait_in(inB, isemB)
            total = compute(inB, 2 * p + 1, total)

            @pl.when(p < PAIRS - 1)
            def _():
                stage_in(2 * p + 3, inB, isemB)

            return total

        total = lax.fori_loop(0, PAIRS, body, jnp.zeros((16,), _f32))
        rsum_v[...] = total
        pltpu.sync_copy(rsum_v, out_h.at[wid, pl.ds(0, 16)])

    return k


def _tc3(u_ref, out_ref):
    s = jnp.sum(u_ref[:, :16], axis=1, keepdims=True)
    out_ref[...] = jnp.sum(s, axis=0, keepdims=True) * (1.0 / (2 * NL))


def _pad1(a, n, val=0):
    return jnp.concatenate([a, jnp.full((n - a.shape[0],), val, a.dtype)])


def kernel(X, W_base, b_base, W_deep, b_deep, reg_W, train_edges, y_train,
           sorted_train_edges, surrogates, y):
    wbs, wbn, wbl = W_base[:DF], W_base[DF:2 * DF], W_base[2 * DF:]
    wds, wdn, wdl = W_deep[:H], W_deep[H:2 * H], W_deep[2 * H:]
    wbl8 = jnp.zeros((8, H), _f32).at[:NLAB].set(wbl)
    wdl8 = jnp.zeros((8, H), _f32).at[:NLAB].set(wdl)
    wa4 = jnp.zeros((H, 4), _f32).at[:, :NLAB].set(reg_W[:H])
    wb4 = jnp.zeros((H, 4), _f32).at[:, :NLAB].set(reg_W[H:])
    bb = b_base.reshape(1, H)
    bd = b_deep.reshape(1, H)

    pad_src = jnp.arange(E_PAD - E, dtype=jnp.int32) % N
    pad_dst = DUMMY + jnp.arange(E_PAD - E, dtype=jnp.int32) % (NP - N)
    src = jnp.concatenate([train_edges[:, 0], pad_src]).reshape(EROWS, CHUNK)
    dst = jnp.concatenate([train_edges[:, 1], pad_dst]).reshape(EROWS, CHUNK)
    ytr = (_pad1(y_train, E_PAD).reshape(EROWS, CHUNK)
           + 8 * jnp.arange(CHUNK, dtype=jnp.int32)[None, :])
    ltab_np = np.zeros((8 * CHUNK, 16), np.float32)
    for kk in range(NLAB):
        rows_k = 8 * np.arange(CHUNK) + kk
        ltab_np[rows_k, 0] = 1.0
        ltab_np[rows_k, 1 + kk] = 1.0
    ltab = jnp.asarray(ltab_np)

    s0 = _pad1(sorted_train_edges[:, :, 0].reshape(-1), NLP)
    s1 = _pad1(sorted_train_edges[:, :, 1].reshape(-1), NLP)
    su = _pad1(surrogates.reshape(-1), NLP)
    yl = _pad1(y[:NL], NLP)
    yn = _pad1(y[NL:], NLP)

    z64 = jnp.zeros((RPT, H), _f32)
    z16 = jnp.zeros((RPT, 16), _f32)

    xwn = pl.pallas_call(
        _tc0,
        grid=(GRID,),
        in_specs=[pl.BlockSpec((RB, DF), lambda i: (i, 0)),
                  pl.BlockSpec((DF, H), lambda i: (0, 0))],
        out_specs=pl.BlockSpec((RB, H), lambda i: (i, 0)),
        out_shape=jax.ShapeDtypeStruct((N, H), _f32),
    )(X, wbn)

    s1p, dlp = _sc_edges(True, 2)(xwn, ltab, src, dst, ytr, z64, z16)

    h1n, pre2, dinv = pl.pallas_call(
        _tc1,
        grid=(GRID,),
        in_specs=[pl.BlockSpec((RB, DF), lambda i: (i, 0)),
                  pl.BlockSpec((DF, H), lambda i: (0, 0)),
                  pl.BlockSpec((RB, 2 * H), lambda i: (i, 0)),
                  pl.BlockSpec((RB, 2 * H), lambda i: (i, 0)),
                  pl.BlockSpec((8, H), lambda i: (0, 0)),
                  pl.BlockSpec((1, H), lambda i: (0, 0)),
                  pl.BlockSpec((H, H), lambda i: (0, 0)),
                  pl.BlockSpec((H, H), lambda i: (0, 0)),
                  pl.BlockSpec((8, H), lambda i: (0, 0)),
                  pl.BlockSpec((1, H), lambda i: (0, 0))],
        out_specs=[pl.BlockSpec((RB, H), lambda i: (i, 0)),
                   pl.BlockSpec((RB, H), lambda i: (i, 0)),
                   pl.BlockSpec((RB, 8), lambda i: (i, 0))],
        out_shape=[jax.ShapeDtypeStruct((N, H), _f32),
                   jax.ShapeDtypeStruct((N, H), _f32),
                   jax.ShapeDtypeStruct((N, 8), _f32)],
    )(X, wbs, s1p, dlp, wbl8, bb, wds, wdn, wdl8, bd)

    (s2p,) = _sc_edges(False, 4)(h1n, src, dst, z64)

    z, ab = pl.pallas_call(
        _tc2,
        grid=(GRID,),
        in_specs=[pl.BlockSpec((RB, H), lambda i: (i, 0)),
                  pl.BlockSpec((RB, 2 * H), lambda i: (i, 0)),
                  pl.BlockSpec((RB, 8), lambda i: (i, 0)),
                  pl.BlockSpec((H, 4), lambda i: (0, 0)),
                  pl.BlockSpec((H, 4), lambda i: (0, 0))],
        out_specs=[pl.BlockSpec((RB, H), lambda i: (i, 0)),
                   pl.BlockSpec((RB, 8), lambda i: (i, 0))],
        out_shape=[jax.ShapeDtypeStruct((N, H), _f32),
                   jax.ShapeDtypeStruct((N, 8), _f32)],
    )(pre2, s2p, dinv, wa4, wb4)

    u4 = _sc_loss()(ab, s0, s1, su, yl, yn)

    tot = pl.pallas_call(
        _tc3,
        grid=(1,),
        in_specs=[pl.BlockSpec((NW, CHUNK), lambda i: (0, 0))],
        out_specs=pl.BlockSpec((1, 1), lambda i: (0, 0)),
        out_shape=jax.ShapeDtypeStruct((1, 1), _f32),
    )(u4)

    return tot[0, 0], z


# speedup vs baseline: 26.5512x; 1.0867x over previous
import functools

import jax
import jax.numpy as jnp
import numpy as np
from jax import lax
from jax.experimental import pallas as pl
from jax.experimental.pallas import tpu as pltpu
from jax.experimental.pallas import tpu_sc as plsc

N = 10000
DF = 128
NLAB = 3
H = 64
E = 320000
EP = 100000
NL = NLAB * EP

NC, NS = 2, 16
NW = NC * NS

NP = 10112
RPT = NP // NS
DUMMY = N

CHUNK = 128
CPT = 80
EPT = CPT * CHUNK
E_PAD = NW * EPT
EROWS = E_PAD // CHUNK

Q = 9600
NLP = NW * Q
LB = 800
NBLK = Q // LB
NCH = LB // 16
PAIRS = NBLK // 2

RB = 2000
GRID = N // RB

_f32 = jnp.float32

_LOG_COEF = (0.6931471805599456, 0.5000009054167627, -0.12500041841408105,
             0.04164875283632623, -0.015616722596026157,
             0.006347559226599455, -0.00264923158324848,
             0.0009052258360685269, -0.00039094821307566503,
             0.00040359603284204326, -0.00018366676407698503)


def _mesh():
    return plsc.VectorSubcoreMesh(core_axis_name="c", subcore_axis_name="s",
                                  num_cores=NC, num_subcores=NS)


_SC_PARAMS = pltpu.CompilerParams(use_tc_tiling_on_sc=False,
                                  needs_layout_passes=False)


def _tc0(x_ref, wn_ref, xwn_ref):
    xwn_ref[...] = jnp.dot(x_ref[...], wn_ref[...],
                           preferred_element_type=_f32)


def _sc_edges(with_lab, K):
    SBLK = CPT // K
    HLF = SBLK // 2
    scratch = [
        pltpu.VMEM((CPT, CHUNK), jnp.int32),
        pltpu.VMEM((CPT, CHUNK), jnp.int32),
        pltpu.VMEM((K * CHUNK, H), _f32),
        pltpu.VMEM((K * CHUNK, H), _f32),
        pltpu.VMEM_SHARED((NP, H), _f32),
        pltpu.SemaphoreType.DMA,
        pltpu.SemaphoreType.DMA,
        pltpu.SemaphoreType.DMA,
        pltpu.SemaphoreType.DMA,
    ]
    out = [jax.ShapeDtypeStruct((NP, 2 * H), _f32)]
    if with_lab:
        scratch += [
            pltpu.VMEM((CPT, CHUNK), jnp.int32),
            pltpu.VMEM((K * CHUNK, 16), _f32),
            pltpu.VMEM((K * CHUNK, 16), _f32),
            pltpu.VMEM_SHARED((NP, 16), _f32),
        ]
        out += [jax.ShapeDtypeStruct((NP, 2 * H), _f32)]

    @functools.partial(pl.kernel, out_type=out, mesh=_mesh(),
                       scratch_types=scratch, compiler_params=_SC_PARAMS)
    def k(*refs):
        if with_lab:
            (tab_h, src_h, dst_h, y_h, z64_h, z16_h,
             s_h, dl_h, src_v, dst_v, rbufA, rbufB, acc,
             gsemA, gsemB, ssemA, ssemB,
             y_v, lbufA, lbufB, acc16) = refs
        else:
            (tab_h, src_h, dst_h, z64_h,
             s_h, src_v, dst_v, rbufA, rbufB, acc,
             gsemA, gsemB, ssemA, ssemB) = refs
            y_v = lbufA = lbufB = acc16 = None
        cid = lax.axis_index("c")
        sid = lax.axis_index("s")
        wid = sid * NC + cid
        rows = pl.ds(sid * RPT, RPT)
        pltpu.sync_copy(z64_h, acc.at[rows])
        if with_lab:
            pltpu.sync_copy(z16_h, acc16.at[rows])
        erows = pl.ds(wid * CPT, CPT)
        pltpu.sync_copy(src_h.at[erows], src_v)
        pltpu.sync_copy(dst_h.at[erows], dst_v)
        if with_lab:
            pltpu.sync_copy(y_h.at[erows], y_v)
        iota16 = lax.iota(jnp.int32, 16)
        if with_lab:
            for lbuf in (lbufA, lbufB):
                pltpu.sync_copy(z16_h.at[pl.ds(0, K * CHUNK)], lbuf)
                for g in range(K * CHUNK // 16):
                    plsc.store_scatter(lbuf,
                                       [g * 16 + iota16, iota16 * 0],
                                       jnp.ones((16,), _f32))
        plsc.subcore_barrier()

        def fill_lab(blk, lbuf):
            for kk in range(K):
                j = blk * K + kk
                for g in range(CHUNK // 16):
                    y16 = y_v[j, pl.ds(g * 16, 16)]
                    rows16 = kk * CHUNK + g * 16 + iota16
                    for lab in range(NLAB):
                        plsc.store_scatter(
                            lbuf, [rows16, iota16 * 0 + (1 + lab)],
                            jnp.where(y16 == lab, 1.0, 0.0).astype(_f32))

        def chunk_slices(rbuf, lbuf, kk):
            sl = pl.ds(kk * CHUNK, CHUNK)
            return rbuf.at[sl], (lbuf.at[sl] if with_lab else None)

        def fire_g(blk, rbuf, lbuf, sem):
            for kk in range(K):
                j = blk * K + kk
                rsl, lsl = chunk_slices(rbuf, lbuf, kk)
                pltpu.async_copy(tab_h.at[src_v.at[j]], rsl, sem)

        def wait_g(rbuf, lbuf, sem):
            for kk in range(K):
                rsl, lsl = chunk_slices(rbuf, lbuf, kk)
                pltpu.make_async_copy(tab_h.at[src_v.at[0]], rsl, sem).wait()

        def fire_s(blk, rbuf, lbuf, sem):
            for kk in range(K):
                j = blk * K + kk
                rsl, lsl = chunk_slices(rbuf, lbuf, kk)
                pltpu.async_copy(rsl, acc.at[dst_v.at[j]], sem, add=True)
                if with_lab:
                    pltpu.async_copy(lsl, acc16.at[dst_v.at[j]], sem,
                                     add=True)

        def wait_s(rbuf, lbuf, sem):
            for kk in range(K):
                rsl, lsl = chunk_slices(rbuf, lbuf, kk)
                pltpu.make_async_copy(rsl, acc.at[dst_v.at[0]], sem).wait()
                if with_lab:
                    pltpu.make_async_copy(lsl, acc16.at[dst_v.at[0]],
                                          sem).wait()

        fire_g(0, rbufA, lbufA, gsemA)
        if with_lab:
            fill_lab(0, lbufA)

        def body(i, carry):
            @pl.when(i > 0)
            def _():
                wait_s(rbufB, lbufB, ssemB)

            fire_g(2 * i + 1, rbufB, lbufB, gsemB)
            if with_lab:
                fill_lab(2 * i + 1, lbufB)
            wait_g(rbufA, lbufA, gsemA)
            fire_s(2 * i, rbufA, lbufA, ssemA)
            wait_g(rbufB, lbufB, gsemB)
            wait_s(rbufA, lbufA, ssemA)

            @pl.when(i < HLF - 1)
            def _():
                fire_g(2 * i + 2, rbufA, lbufA, gsemA)
                if with_lab:
                    fill_lab(2 * i + 2, lbufA)

            fire_s(2 * i + 1, rbufB, lbufB, ssemB)
            return carry

        lax.fori_loop(0, HLF, body, 0)
        wait_s(rbufB, lbufB, ssemB)
        plsc.subcore_barrier()
        pltpu.sync_copy(acc.at[rows], s_h.at[rows, pl.ds(cid * H, H)])
        if with_lab:
            pltpu.sync_copy(acc16.at[rows],
                            dl_h.at[rows, pl.ds(cid * 16, 16)])

    return k


def _tc1(x_ref, wbs_ref, s1_ref, dl_ref, wbl_ref, bb_ref, wds_ref, wdn_ref,
         wdl_ref, bd_ref, h1n_ref, pre2_ref, dinv_ref):
    dot = functools.partial(jnp.dot, preferred_element_type=_f32)
    s1p = s1_ref[...]
    dlp = dl_ref[...]
    s1 = s1p[:, :H] + s1p[:, H:]
    dl = dlp[:, :16] + dlp[:, 16:32]
    di = 1.0 / jnp.maximum(dl[:, 0:1], 1.0)
    lab8 = dl[:, 1:9] * di
    h1 = jnp.tanh(dot(x_ref[...], wbs_ref[...]) + s1 * di
                  + dot(lab8, wbl_ref[...]) + bb_ref[...])
    h1n_ref[...] = dot(h1, wdn_ref[...])
    pre2_ref[...] = (dot(h1, wds_ref[...]) + dot(lab8, wdl_ref[...])
                     + bd_ref[...])
    dinv_ref[...] = jnp.broadcast_to(di, (RB, 8))


def _tc2(pre2_ref, s2_ref, dinv_ref, wa_ref, wb_ref, z_ref, ab_ref):
    dot = functools.partial(jnp.dot, preferred_element_type=_f32)
    s2p = s2_ref[...]
    s2 = s2p[:, :H] + s2p[:, H:]
    z = jnp.tanh(pre2_ref[...] + s2 * dinv_ref[:, 0:1])
    z_ref[...] = z
    ab_ref[...] = jnp.concatenate(
        [dot(z, wa_ref[...]), dot(z, wb_ref[...])], axis=1)


def _logp(u):
    w = u - 2.0
    acc = jnp.full((16,), _LOG_COEF[-1], _f32)
    for c in _LOG_COEF[-2::-1]:
        acc = acc * w + c
    return acc


def _sc_loss():
    scratch = (
        [pltpu.VMEM((N, 8), _f32)]
        + [pltpu.VMEM((5, LB), jnp.int32) for _ in range(2)]
        + [pltpu.VMEM((16,), _f32)]
        + [pltpu.SemaphoreType.DMA for _ in range(2)]
    )

    @functools.partial(
        pl.kernel,
        out_type=jax.ShapeDtypeStruct((NW, CHUNK), _f32),
        mesh=_mesh(),
        scratch_types=scratch,
        compiler_params=_SC_PARAMS)
    def k(ab_h, s0_h, s1_h, su_h, yl_h, yn_h, out_h,
          ab_v, inA, inB, rsum_v, isemA, isemB):
        cid = lax.axis_index("c")
        sid = lax.axis_index("s")
        wid = sid * NC + cid
        pltpu.sync_copy(ab_h, ab_v)
        idx_hs = (s0_h, s1_h, su_h, yl_h, yn_h)

        def stage_in(b, ibuf, sem):
            win = pl.ds(wid * Q + b * LB, LB)
            for r, hh in enumerate(idx_hs):
                pltpu.async_copy(hh.at[win], ibuf.at[r], sem)

        def wait_in(ibuf, sem):
            for r, hh in enumerate(idx_hs):
                pltpu.make_async_copy(hh.at[pl.ds(0, LB)], ibuf.at[r],
                                      sem).wait()

        czero = jnp.zeros((16,), jnp.int32)

        def compute(ibuf, blk, total):
            base = wid * Q + blk * LB

            def ch(i, tot):
                sl = pl.ds(i * 16, 16)
                i0 = ibuf[0, sl]
                i1 = ibuf[1, sl]
                i2 = ibuf[2, sl]
                a0 = plsc.load_gather(ab_v, [i0, czero])
                a1 = plsc.load_gather(ab_v, [i0, czero + 1])
                a2 = plsc.load_gather(ab_v, [i0, czero + 2])
                b0 = plsc.load_gather(ab_v, [i1, czero + 4])
                b1 = plsc.load_gather(ab_v, [i1, czero + 5])
                b2 = plsc.load_gather(ab_v, [i1, czero + 6])
                c0 = plsc.load_gather(ab_v, [i2, czero + 4])
                c1 = plsc.load_gather(ab_v, [i2, czero + 5])
                c2g = plsc.load_gather(ab_v, [i2, czero + 6])

                def nll(p0, p1, p2, yv):
                    m = jnp.maximum(p0, jnp.maximum(p1, p2))
                    u = (jnp.exp(p0 - m) + jnp.exp(p1 - m)
                         + jnp.exp(p2 - m))
                    py = jnp.where(yv == 0, p0, jnp.where(yv == 1, p1, p2))
                    return _logp(u) + m - py

                c_l = nll(a0 + b0, a1 + b1, a2 + b2, ibuf[3, sl])
                c_n = nll(a0 + c0, a1 + c1, a2 + c2g, ibuf[4, sl])
                gidx = base + i * 16 + lax.iota(jnp.int32, 16)
                return tot + jnp.where(gidx < NL, c_l + c_n, 0.0)

            return lax.fori_loop(0, NCH, ch, total)

        stage_in(0, inA, isemA)
        stage_in(1, inB, isemB)

        def body(p, total):
            wait_in(inA, isemA)
            total = compute(inA, 2 * p, total)

            @pl.when(p < PAIRS - 1)
            def _():
                stage_in(2 * p + 2, inA, isemA)

            wait_in(inB, isemB)
            total = compute(inB, 2 * p + 1, total)

            @pl.when(p < PAIRS - 1)
            def _():
                stage_in(2 * p + 3, inB, isemB)

            return total

        total = lax.fori_loop(0, PAIRS, body, jnp.zeros((16,), _f32))
        rsum_v[...] = total
        pltpu.sync_copy(rsum_v, out_h.at[wid, pl.ds(0, 16)])

    return k


def _tc3(u_ref, out_ref):
    s = jnp.sum(u_ref[:, :16], axis=1, keepdims=True)
    out_ref[...] = jnp.sum(s, axis=0, keepdims=True) * (1.0 / (2 * NL))


def _pad1(a, n, val=0):
    return jnp.concatenate([a, jnp.full((n - a.shape[0],), val, a.dtype)])


def kernel(X, W_base, b_base, W_deep, b_deep, reg_W, train_edges, y_train,
           sorted_train_edges, surrogates, y):
    wbs, wbn, wbl = W_base[:DF], W_base[DF:2 * DF], W_base[2 * DF:]
    wds, wdn, wdl = W_deep[:H], W_deep[H:2 * H], W_deep[2 * H:]
    wbl8 = jnp.zeros((8, H), _f32).at[:NLAB].set(wbl)
    wdl8 = jnp.zeros((8, H), _f32).at[:NLAB].set(wdl)
    wa4 = jnp.zeros((H, 4), _f32).at[:, :NLAB].set(reg_W[:H])
    wb4 = jnp.zeros((H, 4), _f32).at[:, :NLAB].set(reg_W[H:])
    bb = b_base.reshape(1, H)
    bd = b_deep.reshape(1, H)

    pad_src = jnp.arange(E_PAD - E, dtype=jnp.int32) % N
    pad_dst = DUMMY + jnp.arange(E_PAD - E, dtype=jnp.int32) % (NP - N)
    src = jnp.concatenate([train_edges[:, 0], pad_src]).reshape(EROWS, CHUNK)
    dst = jnp.concatenate([train_edges[:, 1], pad_dst]).reshape(EROWS, CHUNK)
    ytr = _pad1(y_train, E_PAD).reshape(EROWS, CHUNK)

    s0 = _pad1(sorted_train_edges[:, :, 0].reshape(-1), NLP)
    s1 = _pad1(sorted_train_edges[:, :, 1].reshape(-1), NLP)
    su = _pad1(surrogates.reshape(-1), NLP)
    yl = _pad1(y[:NL], NLP)
    yn = _pad1(y[NL:], NLP)

    z64 = jnp.zeros((RPT, H), _f32)
    z16 = jnp.zeros((RPT, 16), _f32)

    xwn = pl.pallas_call(
        _tc0,
        grid=(GRID,),
        in_specs=[pl.BlockSpec((RB, DF), lambda i: (i, 0)),
                  pl.BlockSpec((DF, H), lambda i: (0, 0))],
        out_specs=pl.BlockSpec((RB, H), lambda i: (i, 0)),
        out_shape=jax.ShapeDtypeStruct((N, H), _f32),
    )(X, wbn)

    s1p, dlp = _sc_edges(True, 2)(xwn, src, dst, ytr, z64, z16)

    h1n, pre2, dinv = pl.pallas_call(
        _tc1,
        grid=(GRID,),
        in_specs=[pl.BlockSpec((RB, DF), lambda i: (i, 0)),
                  pl.BlockSpec((DF, H), lambda i: (0, 0)),
                  pl.BlockSpec((RB, 2 * H), lambda i: (i, 0)),
                  pl.BlockSpec((RB, 2 * H), lambda i: (i, 0)),
                  pl.BlockSpec((8, H), lambda i: (0, 0)),
                  pl.BlockSpec((1, H), lambda i: (0, 0)),
                  pl.BlockSpec((H, H), lambda i: (0, 0)),
                  pl.BlockSpec((H, H), lambda i: (0, 0)),
                  pl.BlockSpec((8, H), lambda i: (0, 0)),
                  pl.BlockSpec((1, H), lambda i: (0, 0))],
        out_specs=[pl.BlockSpec((RB, H), lambda i: (i, 0)),
                   pl.BlockSpec((RB, H), lambda i: (i, 0)),
                   pl.BlockSpec((RB, 8), lambda i: (i, 0))],
        out_shape=[jax.ShapeDtypeStruct((N, H), _f32),
                   jax.ShapeDtypeStruct((N, H), _f32),
                   jax.ShapeDtypeStruct((N, 8), _f32)],
    )(X, wbs, s1p, dlp, wbl8, bb, wds, wdn, wdl8, bd)

    (s2p,) = _sc_edges(False, 4)(h1n, src, dst, z64)

    z, ab = pl.pallas_call(
        _tc2,
        grid=(GRID,),
        in_specs=[pl.BlockSpec((RB, H), lambda i: (i, 0)),
                  pl.BlockSpec((RB, 2 * H), lambda i: (i, 0)),
                  pl.BlockSpec((RB, 8), lambda i: (i, 0)),
                  pl.BlockSpec((H, 4), lambda i: (0, 0)),
                  pl.BlockSpec((H, 4), lambda i: (0, 0))],
        out_specs=[pl.BlockSpec((RB, H), lambda i: (i, 0)),
                   pl.BlockSpec((RB, 8), lambda i: (i, 0))],
        out_shape=[jax.ShapeDtypeStruct((N, H), _f32),
                   jax.ShapeDtypeStruct((N, 8), _f32)],
    )(pre2, s2p, dinv, wa4, wb4)

    u4 = _sc_loss()(ab, s0, s1, su, yl, yn)

    tot = pl.pallas_call(
        _tc3,
        grid=(1,),
        in_specs=[pl.BlockSpec((NW, CHUNK), lambda i: (0, 0))],
        out_specs=pl.BlockSpec((1, 1), lambda i: (0, 0)),
        out_shape=jax.ShapeDtypeStruct((1, 1), _f32),
    )(u4)

    return tot[0, 0], z
